# Initial kernel scaffold; baseline (speedup 1.0000x reference)
import jax
import jax.numpy as jnp
from jax.experimental import pallas as pl


def kernel(x, y, z, W, b):
    raise NotImplementedError("write your pallas kernel here")


# speedup vs baseline: 6.1479x; 6.1479x over previous
import functools

import jax
import jax.numpy as jnp
from jax import lax
from jax.experimental import pallas as pl
from jax.experimental.pallas import tpu as pltpu
from jax.experimental.pallas import tpu_sc as plsc

N = 160000
D = 256
NUM_SEG = 10000

_BM = 8000

_NC = 2
_NS = 16
_NW = _NC * _NS
_CHUNK = N // _NW


def _mv_body(x_ref, w_ref, o_ref):
    o_ref[...] = jnp.dot(x_ref[...], w_ref[...],
                         preferred_element_type=jnp.float32)


def _matvec(x, W):
    return pl.pallas_call(
        _mv_body,
        grid=(N // _BM,),
        in_specs=[
            pl.BlockSpec((_BM, D), lambda i: (i, 0)),
            pl.BlockSpec((D, 1), lambda i: (0, 0)),
        ],
        out_specs=pl.BlockSpec((_BM, 1), lambda i: (i, 0)),
        out_shape=jax.ShapeDtypeStruct((N, 1), jnp.float32),
    )(x, W)


_sc_mesh = plsc.VectorSubcoreMesh(
    core_axis_name="c", subcore_axis_name="s",
    num_cores=_NC, num_subcores=_NS)


@functools.partial(
    pl.kernel,
    out_type=jax.ShapeDtypeStruct((_NC, NUM_SEG), jnp.float32),
    mesh=_sc_mesh,
    scratch_types=[
        pltpu.VMEM((_CHUNK,), jnp.float32),
        pltpu.VMEM((_CHUNK,), jnp.int32),
        pltpu.VMEM_SHARED((NUM_SEG,), jnp.float32),
    ],
)
def _segsum(v_hbm, y_hbm, zeros_hbm, out_hbm, v_vmem, y_vmem, acc_shared):
    c = lax.axis_index("c")
    s = lax.axis_index("s")
    wid = c * _NS + s
    base = wid * _CHUNK

    @pl.when(s == 0)
    def _():
        pltpu.sync_copy(zeros_hbm, acc_shared)

    plsc.subcore_barrier()

    pltpu.sync_copy(v_hbm.at[pl.ds(base, _CHUNK)], v_vmem)
    pltpu.sync_copy(y_hbm.at[pl.ds(base, _CHUNK)], y_vmem)

    pltpu.sync_copy(v_vmem, acc_shared.at[y_vmem], add=True)

    plsc.subcore_barrier()

    @pl.when(s == 0)
    def _():
        pltpu.sync_copy(acc_shared, out_hbm.at[c])


def kernel(x, y, z, W, b):
    v = _matvec(x, W)
    seg = jnp.squeeze(y, axis=1).astype(jnp.int32)
    zeros = jnp.zeros((NUM_SEG,), jnp.float32)
    partials = _segsum(jnp.squeeze(v, axis=1), seg, zeros)
    s = partials[0] + partials[1]
    return s[:, None] + b


# speedup vs baseline: 6.1729x; 1.0041x over previous
import functools

import jax
import jax.numpy as jnp
from jax import lax
from jax.experimental import pallas as pl
from jax.experimental.pallas import tpu as pltpu
from jax.experimental.pallas import tpu_sc as plsc

N = 160000
D = 256
NUM_SEG = 10000

_BM = 8000

_NC = 1
_NS = 16
_NW = _NC * _NS
_CHUNK = N // _NW


def _mv_body(x_ref, w_ref, o_ref):
    o_ref[...] = jnp.dot(x_ref[...], w_ref[...],
                         preferred_element_type=jnp.float32)


def _matvec(x, W):
    return pl.pallas_call(
        _mv_body,
        grid=(N // _BM,),
        in_specs=[
            pl.BlockSpec((_BM, D), lambda i: (i, 0)),
            pl.BlockSpec((D, 1), lambda i: (0, 0)),
        ],
        out_specs=pl.BlockSpec((_BM, 1), lambda i: (i, 0)),
        out_shape=jax.ShapeDtypeStruct((N, 1), jnp.float32),
    )(x, W)


_sc_mesh = plsc.VectorSubcoreMesh(
    core_axis_name="c", subcore_axis_name="s",
    num_cores=_NC, num_subcores=_NS)


@functools.partial(
    pl.kernel,
    out_type=jax.ShapeDtypeStruct((NUM_SEG,), jnp.float32),
    mesh=_sc_mesh,
    scratch_types=[
        pltpu.VMEM((_CHUNK,), jnp.float32),
        pltpu.VMEM((_CHUNK,), jnp.int32),
        pltpu.VMEM_SHARED((NUM_SEG,), jnp.float32),
    ],
)
def _segsum(v_hbm, y_hbm, binit_hbm, out_hbm, v_vmem, y_vmem, acc_shared):
    s = lax.axis_index("s")
    base = s * _CHUNK

    @pl.when(s == 0)
    def _():
        pltpu.sync_copy(binit_hbm, acc_shared)

    plsc.subcore_barrier()

    pltpu.sync_copy(v_hbm.at[pl.ds(base, _CHUNK)], v_vmem)
    pltpu.sync_copy(y_hbm.at[pl.ds(base, _CHUNK)], y_vmem)

    pltpu.sync_copy(v_vmem, acc_shared.at[y_vmem], add=True)

    plsc.subcore_barrier()

    @pl.when(s == 0)
    def _():
        pltpu.sync_copy(acc_shared, out_hbm)


def kernel(x, y, z, W, b):
    v = _matvec(x, W)
    seg = jnp.squeeze(y, axis=1).astype(jnp.int32)
    binit = jnp.broadcast_to(b, (NUM_SEG,))
    out = _segsum(jnp.squeeze(v, axis=1), seg, binit)
    return out[:, None]


# speedup vs baseline: 10.0919x; 1.6349x over previous
import functools

import jax
import jax.numpy as jnp
from jax import lax
from jax.experimental import pallas as pl
from jax.experimental.pallas import tpu as pltpu
from jax.experimental.pallas import tpu_sc as plsc

N = 160000
D = 256
NUM_SEG = 10000

_BM = 10240
_G = _BM // 128
_NROW = N // 128

_NC = 2
_NS = 16
_NW = _NC * _NS
_CHUNK = N // _NW


def _mv_body(x_ref, w_ref, o_ref):
    v = jnp.dot(x_ref[...], w_ref[...], preferred_element_type=jnp.float32)
    o_ref[...] = v.reshape(_G, 128)


def _matvec(x, W):
    grid = (N + _BM - 1) // _BM
    return pl.pallas_call(
        _mv_body,
        grid=(grid,),
        in_specs=[
            pl.BlockSpec((_BM, D), lambda i: (i, 0)),
            pl.BlockSpec((D, 1), lambda i: (0, 0)),
        ],
        out_specs=pl.BlockSpec((_G, 128), lambda i: (i, 0)),
        out_shape=jax.ShapeDtypeStruct((_NROW, 128), jnp.float32),
    )(x, W)


_sc_mesh = plsc.VectorSubcoreMesh(
    core_axis_name="c", subcore_axis_name="s",
    num_cores=_NC, num_subcores=_NS)


@functools.partial(
    pl.kernel,
    out_type=jax.ShapeDtypeStruct((_NC, NUM_SEG), jnp.float32),
    mesh=_sc_mesh,
    scratch_types=[
        pltpu.VMEM((_CHUNK,), jnp.float32),
        pltpu.VMEM((_CHUNK,), jnp.int32),
        pltpu.VMEM_SHARED((NUM_SEG,), jnp.float32),
    ],
)
def _segsum(v_hbm, y_hbm, zeros_hbm, out_hbm, v_vmem, y_vmem, acc_shared):
    c = lax.axis_index("c")
    s = lax.axis_index("s")
    base = (c * _NS + s) * _CHUNK

    @pl.when(s == 0)
    def _():
        pltpu.sync_copy(zeros_hbm, acc_shared)

    plsc.subcore_barrier()

    pltpu.sync_copy(v_hbm.at[pl.ds(base, _CHUNK)], v_vmem)
    pltpu.sync_copy(y_hbm.at[pl.ds(base, _CHUNK)], y_vmem)

    pltpu.sync_copy(v_vmem, acc_shared.at[y_vmem], add=True)

    plsc.subcore_barrier()

    @pl.when(s == 0)
    def _():
        pltpu.sync_copy(acc_shared, out_hbm.at[c])


def kernel(x, y, z, W, b):
    v2 = _matvec(x, W)
    seg = jnp.squeeze(y, axis=1).astype(jnp.int32)
    zeros = jnp.zeros((NUM_SEG,), jnp.float32)
    p = _segsum(v2.reshape(N), seg, zeros)
    return (p[0] + p[1] + b)[:, None]
